# baseline (speedup 1.0000x reference)
import functools

import jax
import jax.numpy as jnp
from jax import lax
from jax.experimental import pallas as pl
from jax.experimental.pallas import tpu as pltpu
from jax.experimental.pallas import tpu_sc as plsc

N_NODES = 10000
N_EDGES = 320000

NC = 2
NS = 16
NW = NC * NS
LANES = 16
WIN = 128
EDGES_PER_TILE = N_EDGES // NW
CHUNKS = EDGES_PER_TILE // LANES
UNROLL = 16
SUP = 10112
N_PAD = 10240

R_MIN = 0.2
R12_CUTOFF = 0.8
INV_WIDTH = 10.0


def _edge_v(l, rmax):
    r = jnp.maximum(l, R_MIN)
    x = jnp.clip(r / rmax, 0.0, 1.0)
    y = 1.0 - x
    y2 = y * y
    y3 = y2 * y
    cut = y3 * y3
    r2 = r * r
    r4 = r2 * r2
    r8 = r4 * r4
    r12 = r8 * r4
    t = jnp.clip((R12_CUTOFF - r) * INV_WIDTH, 0.0, 1.0)
    sm = t * t * (3.0 - 2.0 * t)
    return cut / r12 * sm * 0.25


def _sc_partials(lengths, edge_index, rmax_v):
    mesh = plsc.VectorSubcoreMesh(core_axis_name="c", subcore_axis_name="s")

    @functools.partial(
        pl.kernel,
        out_type=jax.ShapeDtypeStruct((NW * N_PAD,), jnp.float32),
        mesh=mesh,
        scratch_types=[
            pltpu.VMEM((EDGES_PER_TILE,), jnp.float32),
            pltpu.VMEM((2, SUP), jnp.int32),
            pltpu.VMEM((LANES,), jnp.float32),
            pltpu.VMEM((N_PAD,), jnp.float32),
            pltpu.SemaphoreType.DMA,
        ],
        compiler_params=pltpu.CompilerParams(needs_layout_passes=False),
    )
    def k(len_hbm, ei_hbm, rmax_hbm, out_hbm,
          len_v, sidi_v, rm_v, acc_v, sem):
        c = lax.axis_index("c")
        s = lax.axis_index("s")
        wid = c * NS + s
        base = wid * EDGES_PER_TILE
        off = base % WIN
        start = pl.multiple_of(base - off, WIN)

        din = pltpu.async_copy(len_hbm.at[pl.ds(base, EDGES_PER_TILE)],
                               len_v, sem)
        dei = pltpu.async_copy(ei_hbm.at[:, pl.ds(start, SUP)], sidi_v, sem)
        pltpu.sync_copy(rmax_hbm, rm_v)

        @pl.loop(0, N_PAD // LANES)
        def _(i):
            acc_v[pl.ds(i * LANES, LANES)] = jnp.zeros((LANES,), jnp.float32)

        din.wait()
        dei.wait()

        rmax = rm_v[...]

        def do_chunk(i):
            o = i * LANES
            v = _edge_v(len_v[pl.ds(o, LANES)], rmax)
            si = sidi_v[0, pl.ds(off + o, LANES)]
            di = sidi_v[1, pl.ds(off + o, LANES)]
            plsc.addupdate_scatter(acc_v, [si], v)
            plsc.addupdate_scatter(acc_v, [di], v)

        @pl.loop(0, CHUNKS // UNROLL)
        def _(g):
            vs, sis, dis = [], [], []
            for u in range(UNROLL):
                o = (g * UNROLL + u) * LANES
                vs.append(_edge_v(len_v[pl.ds(o, LANES)], rmax))
                sis.append(sidi_v[0, pl.ds(off + o, LANES)])
                dis.append(sidi_v[1, pl.ds(off + o, LANES)])
            for u in range(UNROLL):
                plsc.addupdate_scatter(acc_v, [sis[u]], vs[u])
                plsc.addupdate_scatter(acc_v, [dis[u]], vs[u])

        @pl.loop(CHUNKS - CHUNKS % UNROLL, CHUNKS)
        def _(i):
            do_chunk(i)

        pltpu.sync_copy(acc_v, out_hbm.at[pl.ds(wid * N_PAD, N_PAD)])

    return k(lengths, edge_index, rmax_v)


def _tc_combine(partials_2d):
    def body(p_ref, o_ref):
        o_ref[...] = jnp.sum(p_ref[...].reshape(NW, N_PAD // 128, 128), axis=0)

    return pl.pallas_call(
        body,
        out_shape=jax.ShapeDtypeStruct((N_PAD // 128, 128), jnp.float32),
    )(partials_2d)


def kernel(lengths, node_attrs, edge_index, atomic_numbers, r_max):
    del node_attrs, atomic_numbers
    rmax_v = jnp.broadcast_to(r_max.astype(jnp.float32), (LANES,))
    partials = _sc_partials(lengths.astype(jnp.float32),
                            edge_index.astype(jnp.int32), rmax_v)
    return _tc_combine(partials.reshape(-1, 128)).reshape(-1)[:N_NODES]


# speedup vs baseline: 1.0785x; 1.0785x over previous
import functools

import jax
import jax.numpy as jnp
from jax import lax
from jax.experimental import pallas as pl
from jax.experimental.pallas import tpu as pltpu
from jax.experimental.pallas import tpu_sc as plsc

N_NODES = 10000
N_EDGES = 320000

NC = 2
NS = 16
NW = NC * NS
LANES = 16
WIN = 128
EDGES_PER_TILE = N_EDGES // NW
CHUNKS = EDGES_PER_TILE // LANES
UNROLL = 16
SUP = 10112
N_PAD = 10240
EB = 163840

R_MIN = 0.2
R12_CUTOFF = 0.8
INV_WIDTH = 10.0


def _tc_energy(lengths, rmax_11):
    def body(rm_ref, x_ref, o_ref):
        r = jnp.maximum(x_ref[...], R_MIN)
        x = jnp.clip(r * (1.0 / rm_ref[0, 0]), 0.0, 1.0)
        y = 1.0 - x
        y2 = y * y
        y3 = y2 * y
        cut = y3 * y3
        r2 = r * r
        r4 = r2 * r2
        r8 = r4 * r4
        r12 = r8 * r4
        t = jnp.clip((R12_CUTOFF - r) * INV_WIDTH, 0.0, 1.0)
        sm = t * t * (3.0 - 2.0 * t)
        o_ref[...] = cut / r12 * sm * 0.25

    return pl.pallas_call(
        body,
        grid=(pl.cdiv(N_EDGES, EB),),
        in_specs=[
            pl.BlockSpec(memory_space=pltpu.SMEM),
            pl.BlockSpec((EB,), lambda i: (i,)),
        ],
        out_specs=pl.BlockSpec((EB,), lambda i: (i,)),
        out_shape=jax.ShapeDtypeStruct((N_EDGES,), jnp.float32),
    )(rmax_11, lengths)


def _sc_partials(qv_all, edge_index):
    mesh = plsc.VectorSubcoreMesh(core_axis_name="c", subcore_axis_name="s")

    @functools.partial(
        pl.kernel,
        out_type=jax.ShapeDtypeStruct((NW * N_PAD,), jnp.float32),
        mesh=mesh,
        scratch_types=[
            pltpu.VMEM((EDGES_PER_TILE,), jnp.float32),
            pltpu.VMEM((2, SUP), jnp.int32),
            pltpu.VMEM((N_PAD,), jnp.float32),
            pltpu.SemaphoreType.DMA,
        ],
        compiler_params=pltpu.CompilerParams(needs_layout_passes=False),
    )
    def k(qv_hbm, ei_hbm, out_hbm, qv_v, sidi_v, acc_v, sem):
        c = lax.axis_index("c")
        s = lax.axis_index("s")
        wid = c * NS + s
        base = wid * EDGES_PER_TILE
        off = base % WIN
        start = pl.multiple_of(base - off, WIN)

        dqv = pltpu.async_copy(qv_hbm.at[pl.ds(base, EDGES_PER_TILE)],
                               qv_v, sem)
        dei = pltpu.async_copy(ei_hbm.at[:, pl.ds(start, SUP)], sidi_v, sem)

        @pl.loop(0, N_PAD // LANES)
        def _(i):
            acc_v[pl.ds(i * LANES, LANES)] = jnp.zeros((LANES,), jnp.float32)

        dqv.wait()
        dei.wait()

        def do_chunk(i):
            o = i * LANES
            v = qv_v[pl.ds(o, LANES)]
            si = sidi_v[0, pl.ds(off + o, LANES)]
            di = sidi_v[1, pl.ds(off + o, LANES)]
            plsc.addupdate_scatter(acc_v, [si], v)
            plsc.addupdate_scatter(acc_v, [di], v)

        @pl.loop(0, CHUNKS // UNROLL)
        def _(g):
            vs, sis, dis = [], [], []
            for u in range(UNROLL):
                o = (g * UNROLL + u) * LANES
                vs.append(qv_v[pl.ds(o, LANES)])
                sis.append(sidi_v[0, pl.ds(off + o, LANES)])
                dis.append(sidi_v[1, pl.ds(off + o, LANES)])
            for u in range(UNROLL):
                plsc.addupdate_scatter(acc_v, [sis[u]], vs[u])
                plsc.addupdate_scatter(acc_v, [dis[u]], vs[u])

        @pl.loop(CHUNKS - CHUNKS % UNROLL, CHUNKS)
        def _(i):
            do_chunk(i)

        pltpu.sync_copy(acc_v, out_hbm.at[pl.ds(wid * N_PAD, N_PAD)])

    return k(qv_all, edge_index)


def _tc_combine(partials_2d):
    def body(p_ref, o_ref):
        o_ref[...] = jnp.sum(p_ref[...].reshape(NW, N_PAD // 128, 128), axis=0)

    return pl.pallas_call(
        body,
        out_shape=jax.ShapeDtypeStruct((N_PAD // 128, 128), jnp.float32),
    )(partials_2d)


def kernel(lengths, node_attrs, edge_index, atomic_numbers, r_max):
    del node_attrs, atomic_numbers
    qv_all = _tc_energy(lengths.astype(jnp.float32),
                        r_max.astype(jnp.float32).reshape(1, 1))
    partials = _sc_partials(qv_all, edge_index.astype(jnp.int32))
    return _tc_combine(partials.reshape(-1, 128)).reshape(-1)[:N_NODES]
